# Initial kernel scaffold; baseline (speedup 1.0000x reference)
import jax
import jax.numpy as jnp
from jax.experimental import pallas as pl


def kernel(node_features, edge_features, params, senders, receivers):
    raise NotImplementedError("write your pallas kernel here")


# speedup vs baseline: 1.7203x; 1.7203x over previous
import functools

import jax
import jax.numpy as jnp
from jax import lax
from jax.experimental import pallas as pl
from jax.experimental.pallas import tpu as pltpu
from jax.experimental.pallas import tpu_sc as plsc

N_NODES = 50000
N_EDGES = 800000
LATENT = 64
NC = 2
NS = 16
NW = NC * NS
HALF = N_NODES // NC
DUMMY = HALF
SH_ROWS = HALF + 16
EPW = N_EDGES // NW
CG = 128
N_CHUNK = EPW // CG
TAIL = EPW - N_CHUNK * CG
EPS = N_EDGES // NS
N_CHUNK_S = EPS // CG
TAIL_S = EPS - N_CHUNK_S * CG

_MESH = plsc.VectorSubcoreMesh(
    core_axis_name="c", subcore_axis_name="s", num_cores=NC, num_subcores=NS
)
_SC_PARAMS = pltpu.CompilerParams(use_tc_tiling_on_sc=False)


@functools.partial(
    pl.kernel,
    out_type=(
        jax.ShapeDtypeStruct((N_EDGES, LATENT), jnp.float32),
        jax.ShapeDtypeStruct((N_EDGES, LATENT), jnp.float32),
    ),
    mesh=_MESH,
    scratch_types=[
        pltpu.VMEM((CG,), jnp.int32),
        pltpu.VMEM((CG,), jnp.int32),
        pltpu.VMEM((CG, LATENT), jnp.float32),
        pltpu.VMEM((CG, LATENT), jnp.float32),
        pltpu.SemaphoreType.DMA,
        pltpu.SemaphoreType.DMA,
    ],
    compiler_params=_SC_PARAMS,
)
def _sc_gather(node_hbm, snd_hbm, rcv_hbm, gs_hbm, gr_hbm,
               sidx_v, ridx_v, srow_v, rrow_v, sem_s, sem_r):
    wid = lax.axis_index("s") * NC + lax.axis_index("c")
    wbase = wid * EPW

    def do_chunk(base, n):
        base = pl.multiple_of(base, 8)
        pltpu.sync_copy(snd_hbm.at[pl.ds(base, n)], sidx_v.at[pl.ds(0, n)])
        pltpu.sync_copy(rcv_hbm.at[pl.ds(base, n)], ridx_v.at[pl.ds(0, n)])
        cs = pltpu.async_copy(node_hbm.at[sidx_v.at[pl.ds(0, n)]],
                              srow_v.at[pl.ds(0, n)], sem_s)
        cr = pltpu.async_copy(node_hbm.at[ridx_v.at[pl.ds(0, n)]],
                              rrow_v.at[pl.ds(0, n)], sem_r)
        cs.wait()
        cr.wait()
        pltpu.sync_copy(srow_v.at[pl.ds(0, n)], gs_hbm.at[pl.ds(base, n)])
        pltpu.sync_copy(rrow_v.at[pl.ds(0, n)], gr_hbm.at[pl.ds(base, n)])

    def body(i, _):
        do_chunk(wbase + i * CG, CG)
        return ()

    lax.fori_loop(0, N_CHUNK, body, (), unroll=False)
    do_chunk(wbase + N_CHUNK * CG, TAIL)


@functools.partial(
    pl.kernel,
    out_type=jax.ShapeDtypeStruct((N_NODES, LATENT), jnp.float32),
    mesh=_MESH,
    scratch_types=[
        pltpu.VMEM((CG,), jnp.int32),
        pltpu.VMEM((CG,), jnp.int32),
        pltpu.VMEM((CG, LATENT), jnp.float32),
        pltpu.VMEM_SHARED((SH_ROWS, LATENT), jnp.float32),
    ],
    compiler_params=_SC_PARAMS,
)
def _sc_segsum(edge_hbm, rcv_hbm, zeros_hbm, agg_hbm,
               ridx_v, lidx_v, row_v, acc_sh):
    cid = lax.axis_index("c")
    sid = lax.axis_index("s")
    lo = cid * HALF

    zrows = SH_ROWS // NS
    pltpu.sync_copy(zeros_hbm.at[pl.ds(sid * zrows, zrows)],
                    acc_sh.at[pl.ds(sid * zrows, zrows)])
    plsc.subcore_barrier()

    wbase = sid * EPS

    def do_chunk(base, n):
        base = pl.multiple_of(base, 8)
        pltpu.sync_copy(rcv_hbm.at[pl.ds(base, n)], ridx_v.at[pl.ds(0, n)])
        for k in range((n + 15) // 16):
            r = ridx_v[pl.ds(k * 16, 16)]
            inr = (r >= lo) & (r < lo + HALF)
            lidx_v[pl.ds(k * 16, 16)] = jnp.where(inr, r - lo, DUMMY)
        pltpu.sync_copy(edge_hbm.at[pl.ds(base, n)], row_v.at[pl.ds(0, n)])
        pltpu.sync_copy(row_v.at[pl.ds(0, n)],
                        acc_sh.at[lidx_v.at[pl.ds(0, n)]], add=True)

    def body(i, _):
        do_chunk(wbase + i * CG, CG)
        return ()

    lax.fori_loop(0, N_CHUNK_S, body, (), unroll=False)
    do_chunk(wbase + N_CHUNK_S * CG, TAIL_S)

    plsc.subcore_barrier()

    orows = HALF // NS
    rem = HALF - orows * NS
    pltpu.sync_copy(acc_sh.at[pl.ds(sid * orows, orows)],
                    agg_hbm.at[pl.ds(lo + sid * orows, orows)])

    @pl.when(sid == 0)
    def _():
        pltpu.sync_copy(acc_sh.at[pl.ds(NS * orows, rem)],
                        agg_hbm.at[pl.ds(lo + NS * orows, rem)])


def _ln(y, g, b):
    m = jnp.mean(y, axis=-1, keepdims=True)
    v = jnp.mean((y - m) ** 2, axis=-1, keepdims=True)
    return (y - m) * lax.rsqrt(v + 1e-5) * g + b


def _enc_body(x_ref, mu_ref, sd_ref, w0_ref, b0_ref, w1_ref, b1_ref,
              g_ref, bb_ref, o_ref):
    x = (x_ref[...] - mu_ref[...]) / sd_ref[...]
    h = jnp.maximum(jnp.dot(x, w0_ref[...], preferred_element_type=jnp.float32)
                    + b0_ref[...], 0.0)
    y = jnp.dot(h, w1_ref[...], preferred_element_type=jnp.float32) + b1_ref[...]
    o_ref[...] = _ln(y, g_ref[...], bb_ref[...])


def _encoder(x, mu, sd, mlp, ln, blk):
    n, din = x.shape
    grid = n // blk
    full = lambda a, b: pl.BlockSpec((a, b), lambda i: (0, 0))
    return pl.pallas_call(
        _enc_body,
        grid=(grid,),
        in_specs=[
            pl.BlockSpec((blk, din), lambda i: (i, 0)),
            full(1, din), full(1, din),
            full(din, LATENT), full(1, LATENT),
            full(LATENT, LATENT), full(1, LATENT),
            full(1, LATENT), full(1, LATENT),
        ],
        out_specs=pl.BlockSpec((blk, LATENT), lambda i: (i, 0)),
        out_shape=jax.ShapeDtypeStruct((n, LATENT), jnp.float32),
    )(x, mu.reshape(1, -1), sd.reshape(1, -1),
      mlp["W0"], mlp["b0"].reshape(1, -1), mlp["W1"], mlp["b1"].reshape(1, -1),
      ln["g"].reshape(1, -1), ln["b"].reshape(1, -1))


def _mp_body(a_ref, b_ref, c_ref, w0_ref, b0_ref, w1_ref, b1_ref,
             g_ref, bb_ref, o_ref, *, nin):
    parts = [a_ref[...], b_ref[...], c_ref[...]][:nin]
    x = jnp.concatenate(parts, axis=-1)
    h = jnp.maximum(jnp.dot(x, w0_ref[...], preferred_element_type=jnp.float32)
                    + b0_ref[...], 0.0)
    y = jnp.dot(h, w1_ref[...], preferred_element_type=jnp.float32) + b1_ref[...]
    o_ref[...] = _ln(y, g_ref[...], bb_ref[...]) + a_ref[...]


def _mp_mlp(inputs, mlp, ln, blk):
    nin = len(inputs)
    n = inputs[0].shape[0]
    din = nin * LATENT
    grid = n // blk
    full = lambda a, b: pl.BlockSpec((a, b), lambda i: (0, 0))
    while len(inputs) < 3:
        inputs = inputs + [inputs[0]]
    return pl.pallas_call(
        functools.partial(_mp_body, nin=nin),
        grid=(grid,),
        in_specs=[
            pl.BlockSpec((blk, LATENT), lambda i: (i, 0)),
            pl.BlockSpec((blk, LATENT), lambda i: (i, 0)),
            pl.BlockSpec((blk, LATENT), lambda i: (i, 0)),
            full(din, LATENT), full(1, LATENT),
            full(LATENT, LATENT), full(1, LATENT),
            full(1, LATENT), full(1, LATENT),
        ],
        out_specs=pl.BlockSpec((blk, LATENT), lambda i: (i, 0)),
        out_shape=jax.ShapeDtypeStruct((n, LATENT), jnp.float32),
    )(*inputs, mlp["W0"], mlp["b0"].reshape(1, -1),
      mlp["W1"], mlp["b1"].reshape(1, -1),
      ln["g"].reshape(1, -1), ln["b"].reshape(1, -1))


def _dec_body(x_ref, w0_ref, b0_ref, w1_ref, b1_ref, o_ref):
    h = jnp.maximum(
        jnp.dot(x_ref[...], w0_ref[...], preferred_element_type=jnp.float32)
        + b0_ref[...], 0.0)
    o_ref[...] = jnp.dot(h, w1_ref[...], preferred_element_type=jnp.float32) \
        + b1_ref[...]


def _decoder(x, mlp, blk):
    n = x.shape[0]
    dout = mlp["W1"].shape[1]
    grid = n // blk
    full = lambda a, b: pl.BlockSpec((a, b), lambda i: (0, 0))
    return pl.pallas_call(
        _dec_body,
        grid=(grid,),
        in_specs=[
            pl.BlockSpec((blk, LATENT), lambda i: (i, 0)),
            full(LATENT, LATENT), full(1, LATENT),
            full(LATENT, dout), full(1, dout),
        ],
        out_specs=pl.BlockSpec((blk, dout), lambda i: (i, 0)),
        out_shape=jax.ShapeDtypeStruct((n, dout), jnp.float32),
    )(x, mlp["W0"], mlp["b0"].reshape(1, -1), mlp["W1"],
      mlp["b1"].reshape(1, -1))


def kernel(node_features, edge_features, params, senders, receivers):
    node = _encoder(node_features, params["node_norm_mean"],
                    params["node_norm_std"], params["node_enc"]["mlp"],
                    params["node_enc"]["ln"], blk=2000)
    edge = _encoder(edge_features, params["edge_norm_mean"],
                    params["edge_norm_std"], params["edge_enc"]["mlp"],
                    params["edge_enc"]["ln"], blk=2000)

    zeros = jnp.zeros((SH_ROWS, LATENT), jnp.float32)

    for step in params["steps"]:
        gs, gr = _sc_gather(node, senders, receivers)
        edge = _mp_mlp([edge, gs, gr], step["edge"]["mlp"], step["edge"]["ln"],
                       blk=2000)
        agg = _sc_segsum(edge, receivers, zeros)
        node = _mp_mlp([node, agg], step["node"]["mlp"], step["node"]["ln"],
                       blk=2000)

    return _decoder(node, params["decoder"], blk=2000)


# speedup vs baseline: 1.8743x; 1.0895x over previous
import functools

import jax
import jax.numpy as jnp
from jax import lax
from jax.experimental import pallas as pl
from jax.experimental.pallas import tpu as pltpu
from jax.experimental.pallas import tpu_sc as plsc

N_NODES = 50000
N_EDGES = 800000
LATENT = 64
NC = 2
NS = 16
NW = NC * NS
HALF = N_NODES // NC
DUMMY = HALF
SH_ROWS = HALF + 16
EPW = N_EDGES // NW
CG_G = 1000
N_CHUNK_G = EPW // CG_G
TAIL_G = EPW - N_CHUNK_G * CG_G
CG = 128
EPS = N_EDGES // NS
N_CHUNK_S = EPS // CG
TAIL_S = EPS - N_CHUNK_S * CG

_MESH = plsc.VectorSubcoreMesh(
    core_axis_name="c", subcore_axis_name="s", num_cores=NC, num_subcores=NS
)
_SC_PARAMS = pltpu.CompilerParams(use_tc_tiling_on_sc=False)


@functools.partial(
    pl.kernel,
    out_type=(
        jax.ShapeDtypeStruct((N_EDGES, LATENT), jnp.float32),
        jax.ShapeDtypeStruct((N_EDGES, LATENT), jnp.float32),
    ),
    mesh=_MESH,
    scratch_types=[
        pltpu.VMEM((CG_G,), jnp.int32),
        pltpu.VMEM((CG_G,), jnp.int32),
        pltpu.VMEM((CG_G, LATENT), jnp.float32),
        pltpu.VMEM((CG_G, LATENT), jnp.float32),
        pltpu.SemaphoreType.DMA,
        pltpu.SemaphoreType.DMA,
    ],
    compiler_params=_SC_PARAMS,
)
def _sc_gather(node_hbm, snd_hbm, rcv_hbm, gs_hbm, gr_hbm,
               sidx_v, ridx_v, srow_v, rrow_v, sem_s, sem_r):
    wid = lax.axis_index("s") * NC + lax.axis_index("c")
    wbase = wid * EPW

    def do_chunk(base, n):
        base = pl.multiple_of(base, 8)
        pltpu.sync_copy(snd_hbm.at[pl.ds(base, n)], sidx_v.at[pl.ds(0, n)])
        pltpu.sync_copy(rcv_hbm.at[pl.ds(base, n)], ridx_v.at[pl.ds(0, n)])
        cs = pltpu.async_copy(node_hbm.at[sidx_v.at[pl.ds(0, n)]],
                              srow_v.at[pl.ds(0, n)], sem_s)
        cr = pltpu.async_copy(node_hbm.at[ridx_v.at[pl.ds(0, n)]],
                              rrow_v.at[pl.ds(0, n)], sem_r)
        cs.wait()
        cr.wait()
        pltpu.sync_copy(srow_v.at[pl.ds(0, n)], gs_hbm.at[pl.ds(base, n)])
        pltpu.sync_copy(rrow_v.at[pl.ds(0, n)], gr_hbm.at[pl.ds(base, n)])

    def body(i, _):
        do_chunk(wbase + i * CG_G, CG_G)
        return ()

    lax.fori_loop(0, N_CHUNK_G, body, (), unroll=False)
    if TAIL_G:
        do_chunk(wbase + N_CHUNK_G * CG_G, TAIL_G)


@functools.partial(
    pl.kernel,
    out_type=jax.ShapeDtypeStruct((N_NODES, LATENT), jnp.float32),
    mesh=_MESH,
    scratch_types=[
        pltpu.VMEM((CG,), jnp.int32),
        pltpu.VMEM((CG,), jnp.int32),
        pltpu.VMEM((CG, LATENT), jnp.float32),
        pltpu.VMEM_SHARED((SH_ROWS, LATENT), jnp.float32),
    ],
    compiler_params=_SC_PARAMS,
)
def _sc_segsum(edge_hbm, rcv_hbm, zeros_hbm, agg_hbm,
               ridx_v, lidx_v, row_v, acc_sh):
    cid = lax.axis_index("c")
    sid = lax.axis_index("s")
    lo = cid * HALF

    zrows = SH_ROWS // NS
    pltpu.sync_copy(zeros_hbm.at[pl.ds(sid * zrows, zrows)],
                    acc_sh.at[pl.ds(sid * zrows, zrows)])
    plsc.subcore_barrier()

    wbase = sid * EPS

    def do_chunk(base, n):
        base = pl.multiple_of(base, 8)
        pltpu.sync_copy(rcv_hbm.at[pl.ds(base, n)], ridx_v.at[pl.ds(0, n)])
        for k in range((n + 15) // 16):
            r = ridx_v[pl.ds(k * 16, 16)]
            inr = (r >= lo) & (r < lo + HALF)
            lidx_v[pl.ds(k * 16, 16)] = jnp.where(inr, r - lo, DUMMY)
        pltpu.sync_copy(edge_hbm.at[pl.ds(base, n)], row_v.at[pl.ds(0, n)])
        pltpu.sync_copy(row_v.at[pl.ds(0, n)],
                        acc_sh.at[lidx_v.at[pl.ds(0, n)]], add=True)

    def body(i, _):
        do_chunk(wbase + i * CG, CG)
        return ()

    lax.fori_loop(0, N_CHUNK_S, body, (), unroll=False)
    do_chunk(wbase + N_CHUNK_S * CG, TAIL_S)

    plsc.subcore_barrier()

    orows = HALF // NS
    rem = HALF - orows * NS
    pltpu.sync_copy(acc_sh.at[pl.ds(sid * orows, orows)],
                    agg_hbm.at[pl.ds(lo + sid * orows, orows)])

    @pl.when(sid == 0)
    def _():
        pltpu.sync_copy(acc_sh.at[pl.ds(NS * orows, rem)],
                        agg_hbm.at[pl.ds(lo + NS * orows, rem)])


def _ln(y, g, b):
    m = jnp.mean(y, axis=-1, keepdims=True)
    v = jnp.mean((y - m) ** 2, axis=-1, keepdims=True)
    return (y - m) * lax.rsqrt(v + 1e-5) * g + b


def _enc_body(x_ref, mu_ref, sd_ref, w0_ref, b0_ref, w1_ref, b1_ref,
              g_ref, bb_ref, o_ref):
    x = (x_ref[...] - mu_ref[...]) / sd_ref[...]
    h = jnp.maximum(jnp.dot(x, w0_ref[...], preferred_element_type=jnp.float32)
                    + b0_ref[...], 0.0)
    y = jnp.dot(h, w1_ref[...], preferred_element_type=jnp.float32) + b1_ref[...]
    o_ref[...] = _ln(y, g_ref[...], bb_ref[...])


def _encoder(x, mu, sd, mlp, ln, blk):
    n, din = x.shape
    grid = n // blk
    full = lambda a, b: pl.BlockSpec((a, b), lambda i: (0, 0))
    return pl.pallas_call(
        _enc_body,
        grid=(grid,),
        in_specs=[
            pl.BlockSpec((blk, din), lambda i: (i, 0)),
            full(1, din), full(1, din),
            full(din, LATENT), full(1, LATENT),
            full(LATENT, LATENT), full(1, LATENT),
            full(1, LATENT), full(1, LATENT),
        ],
        out_specs=pl.BlockSpec((blk, LATENT), lambda i: (i, 0)),
        out_shape=jax.ShapeDtypeStruct((n, LATENT), jnp.float32),
    )(x, mu.reshape(1, -1), sd.reshape(1, -1),
      mlp["W0"], mlp["b0"].reshape(1, -1), mlp["W1"], mlp["b1"].reshape(1, -1),
      ln["g"].reshape(1, -1), ln["b"].reshape(1, -1))


def _mp_body(a_ref, b_ref, c_ref, w0_ref, b0_ref, w1_ref, b1_ref,
             g_ref, bb_ref, o_ref, *, nin):
    parts = [a_ref[...], b_ref[...], c_ref[...]][:nin]
    x = jnp.concatenate(parts, axis=-1)
    h = jnp.maximum(jnp.dot(x, w0_ref[...], preferred_element_type=jnp.float32)
                    + b0_ref[...], 0.0)
    y = jnp.dot(h, w1_ref[...], preferred_element_type=jnp.float32) + b1_ref[...]
    o_ref[...] = _ln(y, g_ref[...], bb_ref[...]) + a_ref[...]


def _mp_mlp(inputs, mlp, ln, blk):
    nin = len(inputs)
    n = inputs[0].shape[0]
    din = nin * LATENT
    grid = n // blk
    full = lambda a, b: pl.BlockSpec((a, b), lambda i: (0, 0))
    while len(inputs) < 3:
        inputs = inputs + [inputs[0]]
    return pl.pallas_call(
        functools.partial(_mp_body, nin=nin),
        grid=(grid,),
        in_specs=[
            pl.BlockSpec((blk, LATENT), lambda i: (i, 0)),
            pl.BlockSpec((blk, LATENT), lambda i: (i, 0)),
            pl.BlockSpec((blk, LATENT), lambda i: (i, 0)),
            full(din, LATENT), full(1, LATENT),
            full(LATENT, LATENT), full(1, LATENT),
            full(1, LATENT), full(1, LATENT),
        ],
        out_specs=pl.BlockSpec((blk, LATENT), lambda i: (i, 0)),
        out_shape=jax.ShapeDtypeStruct((n, LATENT), jnp.float32),
    )(*inputs, mlp["W0"], mlp["b0"].reshape(1, -1),
      mlp["W1"], mlp["b1"].reshape(1, -1),
      ln["g"].reshape(1, -1), ln["b"].reshape(1, -1))


def _dec_body(x_ref, w0_ref, b0_ref, w1_ref, b1_ref, o_ref):
    h = jnp.maximum(
        jnp.dot(x_ref[...], w0_ref[...], preferred_element_type=jnp.float32)
        + b0_ref[...], 0.0)
    o_ref[...] = jnp.dot(h, w1_ref[...], preferred_element_type=jnp.float32) \
        + b1_ref[...]


def _decoder(x, mlp, blk):
    n = x.shape[0]
    dout = mlp["W1"].shape[1]
    grid = n // blk
    full = lambda a, b: pl.BlockSpec((a, b), lambda i: (0, 0))
    return pl.pallas_call(
        _dec_body,
        grid=(grid,),
        in_specs=[
            pl.BlockSpec((blk, LATENT), lambda i: (i, 0)),
            full(LATENT, LATENT), full(1, LATENT),
            full(LATENT, dout), full(1, dout),
        ],
        out_specs=pl.BlockSpec((blk, dout), lambda i: (i, 0)),
        out_shape=jax.ShapeDtypeStruct((n, dout), jnp.float32),
    )(x, mlp["W0"], mlp["b0"].reshape(1, -1), mlp["W1"],
      mlp["b1"].reshape(1, -1))


def kernel(node_features, edge_features, params, senders, receivers):
    node = _encoder(node_features, params["node_norm_mean"],
                    params["node_norm_std"], params["node_enc"]["mlp"],
                    params["node_enc"]["ln"], blk=2000)
    edge = _encoder(edge_features, params["edge_norm_mean"],
                    params["edge_norm_std"], params["edge_enc"]["mlp"],
                    params["edge_enc"]["ln"], blk=2000)

    zeros = jnp.zeros((SH_ROWS, LATENT), jnp.float32)

    for step in params["steps"]:
        gs, gr = _sc_gather(node, senders, receivers)
        edge = _mp_mlp([edge, gs, gr], step["edge"]["mlp"], step["edge"]["ln"],
                       blk=2000)
        agg = _sc_segsum(edge, receivers, zeros)
        node = _mp_mlp([node, agg], step["node"]["mlp"], step["node"]["ln"],
                       blk=2000)

    return _decoder(node, params["decoder"], blk=2000)
